# Initial kernel scaffold; baseline (speedup 1.0000x reference)
import functools

import jax
import jax.numpy as jnp
from jax import lax
from jax.experimental import pallas as pl
from jax.experimental.pallas import tpu as pltpu
from jax.experimental.pallas import tpu_sc as plsc

N = 10000
NPAD = 10240
C = 64
NC, NS, L = 2, 16, 16
NW = NC * NS
CHUNK = 128
E_TOTAL = 330000
CPT = 81
EPT = CPT * CHUNK
E_PAD = NW * EPT
RPW = NPAD // NS

_MESH = plsc.VectorSubcoreMesh(core_axis_name="c", subcore_axis_name="s")


def _proj_body(x_ref, wl_ref, bl_ref, wr_ref, br_ref, ol_ref, or_ref):
    xb = x_ref[...]
    ol_ref[...] = jnp.dot(xb, wl_ref[...],
                          preferred_element_type=jnp.float32) + bl_ref[...]
    or_ref[...] = jnp.dot(xb, wr_ref[...],
                          preferred_element_type=jnp.float32) + br_ref[...]


def _dense_pair(x, Wl, bl, Wr, br):
    n, f = x.shape
    c = Wl.shape[1]
    blk = 400
    return pl.pallas_call(
        _proj_body,
        grid=(n // blk,),
        in_specs=[
            pl.BlockSpec((blk, f), lambda i: (i, 0)),
            pl.BlockSpec((f, c), lambda i: (0, 0)),
            pl.BlockSpec((1, c), lambda i: (0, 0)),
            pl.BlockSpec((f, c), lambda i: (0, 0)),
            pl.BlockSpec((1, c), lambda i: (0, 0)),
        ],
        out_specs=[
            pl.BlockSpec((blk, c), lambda i: (i, 0)),
            pl.BlockSpec((blk, c), lambda i: (i, 0)),
        ],
        out_shape=[jax.ShapeDtypeStruct((n, c), jnp.float32)] * 2,
    )(x, Wl, bl, Wr, br)


def _relu_proj_body(p0_ref, p1_ref, b_ref, wl_ref, bl_ref, wr_ref, br_ref,
                    ol_ref, or_ref):
    h = jnp.maximum(p0_ref[...] + p1_ref[...] + b_ref[...], 0.0)
    ol_ref[...] = jnp.dot(h, wl_ref[...],
                          preferred_element_type=jnp.float32) + bl_ref[...]
    or_ref[...] = jnp.dot(h, wr_ref[...],
                          preferred_element_type=jnp.float32) + br_ref[...]


def _relu_dense_pair(p0, p1, b, Wl, bl, Wr, br):
    n, f = p0.shape
    c = Wl.shape[1]
    blk = 400
    return pl.pallas_call(
        _relu_proj_body,
        grid=(n // blk,),
        in_specs=[
            pl.BlockSpec((blk, f), lambda i: (i, 0)),
            pl.BlockSpec((blk, f), lambda i: (i, 0)),
            pl.BlockSpec((1, f), lambda i: (0, 0)),
            pl.BlockSpec((f, c), lambda i: (0, 0)),
            pl.BlockSpec((1, c), lambda i: (0, 0)),
            pl.BlockSpec((f, c), lambda i: (0, 0)),
            pl.BlockSpec((1, c), lambda i: (0, 0)),
        ],
        out_specs=[
            pl.BlockSpec((blk, c), lambda i: (i, 0)),
            pl.BlockSpec((blk, c), lambda i: (i, 0)),
        ],
        out_shape=[jax.ShapeDtypeStruct((n, c), jnp.float32)] * 2,
    )(p0, p1, b, Wl, bl, Wr, br)


def _decode_body(p0_ref, p1_ref, b_ref, wd_ref, bd_ref, h_ref, out_ref):
    h = p0_ref[...] + p1_ref[...] + b_ref[...]
    h_ref[...] = h
    out_ref[...] = jnp.dot(h, wd_ref[...],
                           preferred_element_type=jnp.float32) + bd_ref[...]


def _decode(p0, p1, b, Wd, bd):
    n, f = p0.shape
    od = Wd.shape[1]
    blk = 400
    return pl.pallas_call(
        _decode_body,
        grid=(n // blk,),
        in_specs=[
            pl.BlockSpec((blk, f), lambda i: (i, 0)),
            pl.BlockSpec((blk, f), lambda i: (i, 0)),
            pl.BlockSpec((1, f), lambda i: (0, 0)),
            pl.BlockSpec((f, od), lambda i: (0, 0)),
            pl.BlockSpec((1, od), lambda i: (0, 0)),
        ],
        out_specs=[
            pl.BlockSpec((blk, f), lambda i: (i, 0)),
            pl.BlockSpec((blk, od), lambda i: (i, 0)),
        ],
        out_shape=[
            jax.ShapeDtypeStruct((n, f), jnp.float32),
            jax.ShapeDtypeStruct((n, od), jnp.float32),
        ],
    )(p0, p1, b, Wd, bd)


def _sc_logits_body(xl_hbm, xr_hbm, src_hbm, dst_hbm, att_hbm,
                    ex_hbm, dpart_hbm,
                    src_v, dst_v, xlr, xrr, att_v, lbuf, exbuf, zbuf,
                    dsh, sem1, sem2):
    c = lax.axis_index("c")
    s = lax.axis_index("s")
    wid = c * NS + s

    for i in range(RPW // L):
        zbuf[pl.ds(i * L, L)] = jnp.zeros((L,), jnp.float32)
    pltpu.sync_copy(zbuf, dsh.at[pl.ds(s * RPW, RPW)])
    plsc.subcore_barrier()

    pltpu.sync_copy(att_hbm, att_v)
    att_regs = [att_v[pl.ds(16 * k, 16)] for k in range(4)]
    lane = lax.iota(jnp.int32, (16,))
    m15 = lane == 15
    tile_base = wid * EPT

    def chunk_body(ch, carry):
        base = tile_base + ch * CHUNK
        pltpu.sync_copy(src_hbm.at[pl.ds(base, CHUNK)], src_v)
        pltpu.sync_copy(dst_hbm.at[pl.ds(base, CHUNK)], dst_v)
        cp1 = pltpu.async_copy(xl_hbm.at[src_v], xlr, sem1)
        cp2 = pltpu.async_copy(xr_hbm.at[dst_v], xrr, sem2)
        cp1.wait()
        cp2.wait()

        @plsc.parallel_loop(0, CHUNK, unroll=4)
        def _edge(e):
            w = None
            for k in range(4):
                m = xlr[e, pl.ds(16 * k, 16)] + xrr[e, pl.ds(16 * k, 16)]
                t = jnp.maximum(m, 0.2 * m) * att_regs[k]
                w = t if w is None else w + t
            sacc = plsc.cumsum(w)
            eidx = jnp.broadcast_to(e, (16,)).astype(jnp.int32)
            plsc.store_scatter(lbuf, [eidx], sacc, mask=m15)

        for g in range(CHUNK // L):
            lv = lbuf[pl.ds(L * g, L)]
            gidx = base + L * g + lane
            ve = jnp.where(gidx < E_TOTAL,
                           jnp.exp(jnp.minimum(lv, 60.0)), 0.0)
            exbuf[pl.ds(L * g, L)] = ve
        pltpu.sync_copy(exbuf, ex_hbm.at[pl.ds(base, CHUNK)])
        pltpu.sync_copy(exbuf, dsh.at[dst_v], add=True)
        return carry

    lax.fori_loop(0, CPT, chunk_body, 0)
    plsc.subcore_barrier()

    @pl.when(s == 0)
    def _():
        pltpu.sync_copy(dsh, dpart_hbm.at[c])


def _sc_logits(xl, xr, src, dst, att):
    f = pl.kernel(
        _sc_logits_body,
        out_type=[
            jax.ShapeDtypeStruct((E_PAD,), jnp.float32),
            jax.ShapeDtypeStruct((NC, NPAD), jnp.float32),
        ],
        mesh=_MESH,
        scratch_types=[
            pltpu.VMEM((CHUNK,), jnp.int32),
            pltpu.VMEM((CHUNK,), jnp.int32),
            pltpu.VMEM((CHUNK, C), jnp.float32),
            pltpu.VMEM((CHUNK, C), jnp.float32),
            pltpu.VMEM((C,), jnp.float32),
            pltpu.VMEM((CHUNK,), jnp.float32),
            pltpu.VMEM((CHUNK,), jnp.float32),
            pltpu.VMEM((RPW,), jnp.float32),
            pltpu.VMEM_SHARED((NPAD,), jnp.float32),
            pltpu.SemaphoreType.DMA,
            pltpu.SemaphoreType.DMA,
        ],
    )
    return f(xl, xr, src, dst, att)


def _sc_scatter_body(xl_hbm, src_hbm, dst_hbm, ex_hbm, dpart_hbm,
                     hpart_hbm,
                     src_v, dst_v, ex_v, alpha_v, xlr, prod, dsum, dtmp,
                     zrow, osh, sem1):
    c = lax.axis_index("c")
    s = lax.axis_index("s")
    wid = c * NS + s

    for j in range(C):
        for k in range(C // L):
            zrow[j, pl.ds(k * L, L)] = jnp.zeros((L,), jnp.float32)
    for i in range(RPW // C):
        pltpu.sync_copy(zrow, osh.at[pl.ds(s * RPW + i * C, C)])
    plsc.subcore_barrier()

    pltpu.sync_copy(dpart_hbm.at[0], dsum)
    pltpu.sync_copy(dpart_hbm.at[1], dtmp)

    def dsum_body(i, carry):
        sl = pl.ds(i * L, L)
        dsum[sl] = dsum[sl] + dtmp[sl] + 1e-16
        return carry

    lax.fori_loop(0, NPAD // L, dsum_body, 0)

    tile_base = wid * EPT

    def chunk_body(ch, carry):
        base = tile_base + ch * CHUNK
        pltpu.sync_copy(src_hbm.at[pl.ds(base, CHUNK)], src_v)
        pltpu.sync_copy(dst_hbm.at[pl.ds(base, CHUNK)], dst_v)
        pltpu.sync_copy(ex_hbm.at[pl.ds(base, CHUNK)], ex_v)
        cp1 = pltpu.async_copy(xl_hbm.at[src_v], xlr, sem1)
        cp1.wait()

        for g in range(CHUNK // L):
            dsv = dst_v[pl.ds(L * g, L)]
            dv = plsc.load_gather(dsum, [dsv])
            alpha_v[pl.ds(L * g, L)] = ex_v[pl.ds(L * g, L)] / dv

        @plsc.parallel_loop(0, CHUNK, unroll=4)
        def _edge(e):
            eidx = jnp.broadcast_to(e, (16,)).astype(jnp.int32)
            av = plsc.load_gather(alpha_v, [eidx])
            for k in range(4):
                prod[e, pl.ds(16 * k, 16)] = xlr[e, pl.ds(16 * k, 16)] * av

        pltpu.sync_copy(prod, osh.at[dst_v], add=True)
        return carry

    lax.fori_loop(0, CPT, chunk_body, 0)
    plsc.subcore_barrier()
    pltpu.sync_copy(osh.at[pl.ds(s * RPW, RPW)],
                    hpart_hbm.at[c, pl.ds(s * RPW, RPW)])


def _sc_scatter(xl, src, dst, ex, dpart):
    f = pl.kernel(
        _sc_scatter_body,
        out_type=[
            jax.ShapeDtypeStruct((NC, NPAD, C), jnp.float32),
        ],
        mesh=_MESH,
        scratch_types=[
            pltpu.VMEM((CHUNK,), jnp.int32),
            pltpu.VMEM((CHUNK,), jnp.int32),
            pltpu.VMEM((CHUNK,), jnp.float32),
            pltpu.VMEM((CHUNK,), jnp.float32),
            pltpu.VMEM((CHUNK, C), jnp.float32),
            pltpu.VMEM((CHUNK, C), jnp.float32),
            pltpu.VMEM((NPAD,), jnp.float32),
            pltpu.VMEM((NPAD,), jnp.float32),
            pltpu.VMEM((C, C), jnp.float32),
            pltpu.VMEM_SHARED((NPAD, C), jnp.float32),
            pltpu.SemaphoreType.DMA,
        ],
    )
    return f(xl, src, dst, ex, dpart)[0]


def kernel(x, edge_index, Wl1, bl1, Wr1, br1, att1, bias1,
           Wl2, bl2, Wr2, br2, att2, bias2, Wd, bd):
    loop = jnp.arange(N, dtype=jnp.int32)
    src = jnp.concatenate([edge_index[0].astype(jnp.int32), loop])
    dst = jnp.concatenate([edge_index[1].astype(jnp.int32), loop])
    padi = jnp.arange(E_PAD - E_TOTAL, dtype=jnp.int32) % N
    src = jnp.concatenate([src, padi])
    dst = jnp.concatenate([dst, padi])

    att1f = att1.reshape(C)
    att2f = att2.reshape(C)
    r = lambda v: v.reshape(1, -1)

    xl1, xr1 = _dense_pair(x, Wl1, r(bl1), Wr1, r(br1))
    ex1, dp1 = _sc_logits(xl1, xr1, src, dst, att1f)
    hp1 = _sc_scatter(xl1, src, dst, ex1, dp1)

    xl2, xr2 = _relu_dense_pair(hp1[0, :N], hp1[1, :N], r(bias1),
                                Wl2, r(bl2), Wr2, r(br2))
    ex2, dp2 = _sc_logits(xl2, xr2, src, dst, att2f)
    hp2 = _sc_scatter(xl2, src, dst, ex2, dp2)

    h, out = _decode(hp2[0, :N], hp2[1, :N], r(bias2), Wd, r(bd))
    return (out, h)


# speedup vs baseline: 11.9473x; 11.9473x over previous
import functools

import jax
import jax.numpy as jnp
from jax import lax
from jax.experimental import pallas as pl
from jax.experimental.pallas import tpu as pltpu
from jax.experimental.pallas import tpu_sc as plsc

N = 10000
NPAD = 10240
C = 64
NC, NS, L = 2, 16, 16
NW = NC * NS
CHUNK = 128
E_TOTAL = 330000
CPT = 81
EPT = CPT * CHUNK
E_PAD = NW * EPT
RPW = NPAD // NS

_MESH = plsc.VectorSubcoreMesh(core_axis_name="c", subcore_axis_name="s")


def _proj_body(x_ref, wl_ref, bl_ref, wr_ref, br_ref, ol_ref, or_ref):
    xb = x_ref[...]
    ol_ref[...] = jnp.dot(xb, wl_ref[...],
                          preferred_element_type=jnp.float32) + bl_ref[...]
    or_ref[...] = jnp.dot(xb, wr_ref[...],
                          preferred_element_type=jnp.float32) + br_ref[...]


def _dense_pair(x, Wl, bl, Wr, br):
    n, f = x.shape
    c = Wl.shape[1]
    blk = 400
    return pl.pallas_call(
        _proj_body,
        grid=(n // blk,),
        in_specs=[
            pl.BlockSpec((blk, f), lambda i: (i, 0)),
            pl.BlockSpec((f, c), lambda i: (0, 0)),
            pl.BlockSpec((1, c), lambda i: (0, 0)),
            pl.BlockSpec((f, c), lambda i: (0, 0)),
            pl.BlockSpec((1, c), lambda i: (0, 0)),
        ],
        out_specs=[
            pl.BlockSpec((blk, c), lambda i: (i, 0)),
            pl.BlockSpec((blk, c), lambda i: (i, 0)),
        ],
        out_shape=[jax.ShapeDtypeStruct((n, c), jnp.float32)] * 2,
    )(x, Wl, bl, Wr, br)


def _relu_proj_body(p0_ref, p1_ref, b_ref, wl_ref, bl_ref, wr_ref, br_ref,
                    ol_ref, or_ref):
    h = jnp.maximum(p0_ref[...] + p1_ref[...] + b_ref[...], 0.0)
    ol_ref[...] = jnp.dot(h, wl_ref[...],
                          preferred_element_type=jnp.float32) + bl_ref[...]
    or_ref[...] = jnp.dot(h, wr_ref[...],
                          preferred_element_type=jnp.float32) + br_ref[...]


def _relu_dense_pair(p0, p1, b, Wl, bl, Wr, br):
    n, f = p0.shape
    c = Wl.shape[1]
    blk = 400
    return pl.pallas_call(
        _relu_proj_body,
        grid=(n // blk,),
        in_specs=[
            pl.BlockSpec((blk, f), lambda i: (i, 0)),
            pl.BlockSpec((blk, f), lambda i: (i, 0)),
            pl.BlockSpec((1, f), lambda i: (0, 0)),
            pl.BlockSpec((f, c), lambda i: (0, 0)),
            pl.BlockSpec((1, c), lambda i: (0, 0)),
            pl.BlockSpec((f, c), lambda i: (0, 0)),
            pl.BlockSpec((1, c), lambda i: (0, 0)),
        ],
        out_specs=[
            pl.BlockSpec((blk, c), lambda i: (i, 0)),
            pl.BlockSpec((blk, c), lambda i: (i, 0)),
        ],
        out_shape=[jax.ShapeDtypeStruct((n, c), jnp.float32)] * 2,
    )(p0, p1, b, Wl, bl, Wr, br)


def _decode_body(p0_ref, p1_ref, b_ref, wd_ref, bd_ref, h_ref, out_ref):
    h = p0_ref[...] + p1_ref[...] + b_ref[...]
    h_ref[...] = h
    out_ref[...] = jnp.dot(h, wd_ref[...],
                           preferred_element_type=jnp.float32) + bd_ref[...]


def _decode(p0, p1, b, Wd, bd):
    n, f = p0.shape
    od = Wd.shape[1]
    blk = 400
    return pl.pallas_call(
        _decode_body,
        grid=(n // blk,),
        in_specs=[
            pl.BlockSpec((blk, f), lambda i: (i, 0)),
            pl.BlockSpec((blk, f), lambda i: (i, 0)),
            pl.BlockSpec((1, f), lambda i: (0, 0)),
            pl.BlockSpec((f, od), lambda i: (0, 0)),
            pl.BlockSpec((1, od), lambda i: (0, 0)),
        ],
        out_specs=[
            pl.BlockSpec((blk, f), lambda i: (i, 0)),
            pl.BlockSpec((blk, od), lambda i: (i, 0)),
        ],
        out_shape=[
            jax.ShapeDtypeStruct((n, f), jnp.float32),
            jax.ShapeDtypeStruct((n, od), jnp.float32),
        ],
    )(p0, p1, b, Wd, bd)


def _sc_logits_body(xl_hbm, xr_hbm, src_hbm, dst_hbm, att_hbm,
                    ex_hbm, dpart_hbm,
                    src_v, dst_v, xlr, xrr, att_v, lbuf, exbuf, zbuf,
                    dsh, sem1, sem2):
    c = lax.axis_index("c")
    s = lax.axis_index("s")
    wid = c * NS + s

    for i in range(RPW // L):
        zbuf[pl.ds(i * L, L)] = jnp.zeros((L,), jnp.float32)
    pltpu.sync_copy(zbuf, dsh.at[pl.ds(s * RPW, RPW)])
    plsc.subcore_barrier()

    pltpu.sync_copy(att_hbm, att_v)
    att_regs = [att_v[pl.ds(16 * k, 16)] for k in range(4)]
    lane = lax.iota(jnp.int32, 16)
    tile_base = wid * EPT

    def chunk_body(ch, carry):
        base = tile_base + ch * CHUNK
        pltpu.sync_copy(src_hbm.at[pl.ds(base, CHUNK)], src_v)
        pltpu.sync_copy(dst_hbm.at[pl.ds(base, CHUNK)], dst_v)
        cp1 = pltpu.async_copy(xl_hbm.at[src_v], xlr, sem1)
        cp2 = pltpu.async_copy(xr_hbm.at[dst_v], xrr, sem2)
        cp1.wait()
        cp2.wait()

        @plsc.parallel_loop(0, CHUNK, unroll=4)
        def _edge(e):
            w = None
            for k in range(4):
                m = xlr[e, pl.ds(16 * k, 16)] + xrr[e, pl.ds(16 * k, 16)]
                t = jnp.maximum(m, 0.2 * m) * att_regs[k]
                w = t if w is None else w + t
            lbuf[e, :] = w

        for g in range(CHUNK // L):
            rows = L * g + lane
            acc = plsc.load_gather(lbuf, [rows, jnp.full((L,), 0, jnp.int32)])
            for l in range(1, L):
                acc = acc + plsc.load_gather(
                    lbuf, [rows, jnp.full((L,), l, jnp.int32)])
            gidx = base + L * g + lane
            ve = jnp.where(gidx < E_TOTAL,
                           jnp.exp(jnp.minimum(acc, 60.0)), 0.0)
            exbuf[pl.ds(L * g, L)] = ve
        pltpu.sync_copy(exbuf, ex_hbm.at[pl.ds(base, CHUNK)])
        pltpu.sync_copy(exbuf, dsh.at[dst_v], add=True)
        return carry

    lax.fori_loop(0, CPT, chunk_body, 0)
    plsc.subcore_barrier()

    @pl.when(s == 0)
    def _():
        pltpu.sync_copy(dsh, dpart_hbm.at[c])


def _sc_logits(xl, xr, src, dst, att):
    f = pl.kernel(
        _sc_logits_body,
        out_type=[
            jax.ShapeDtypeStruct((E_PAD,), jnp.float32),
            jax.ShapeDtypeStruct((NC, NPAD), jnp.float32),
        ],
        mesh=_MESH,
        compiler_params=pltpu.CompilerParams(
            needs_layout_passes=False, use_tc_tiling_on_sc=False),
        scratch_types=[
            pltpu.VMEM((CHUNK,), jnp.int32),
            pltpu.VMEM((CHUNK,), jnp.int32),
            pltpu.VMEM((CHUNK, C), jnp.float32),
            pltpu.VMEM((CHUNK, C), jnp.float32),
            pltpu.VMEM((C,), jnp.float32),
            pltpu.VMEM((CHUNK, L), jnp.float32),
            pltpu.VMEM((CHUNK,), jnp.float32),
            pltpu.VMEM((RPW,), jnp.float32),
            pltpu.VMEM_SHARED((NPAD,), jnp.float32),
            pltpu.SemaphoreType.DMA,
            pltpu.SemaphoreType.DMA,
        ],
    )
    return f(xl, xr, src, dst, att)


def _sc_scatter_body(xl_hbm, src_hbm, dst_hbm, ex_hbm, dpart_hbm,
                     hpart_hbm,
                     src_v, dst_v, ex_v, alpha_v, xlr, prod, dsum, dtmp,
                     zrow, osh, sem1):
    c = lax.axis_index("c")
    s = lax.axis_index("s")
    wid = c * NS + s

    for j in range(C):
        for k in range(C // L):
            zrow[j, pl.ds(k * L, L)] = jnp.zeros((L,), jnp.float32)
    for i in range(RPW // C):
        pltpu.sync_copy(zrow, osh.at[pl.ds(s * RPW + i * C, C)])
    plsc.subcore_barrier()

    pltpu.sync_copy(dpart_hbm.at[0], dsum)
    pltpu.sync_copy(dpart_hbm.at[1], dtmp)

    def dsum_body(i, carry):
        sl = pl.ds(i * L, L)
        dsum[sl] = dsum[sl] + dtmp[sl] + 1e-16
        return carry

    lax.fori_loop(0, NPAD // L, dsum_body, 0)

    tile_base = wid * EPT

    def chunk_body(ch, carry):
        base = tile_base + ch * CHUNK
        pltpu.sync_copy(src_hbm.at[pl.ds(base, CHUNK)], src_v)
        pltpu.sync_copy(dst_hbm.at[pl.ds(base, CHUNK)], dst_v)
        pltpu.sync_copy(ex_hbm.at[pl.ds(base, CHUNK)], ex_v)
        cp1 = pltpu.async_copy(xl_hbm.at[src_v], xlr, sem1)
        cp1.wait()

        for g in range(CHUNK // L):
            dsv = dst_v[pl.ds(L * g, L)]
            dv = plsc.load_gather(dsum, [dsv])
            alpha_v[pl.ds(L * g, L)] = ex_v[pl.ds(L * g, L)] / dv

        @plsc.parallel_loop(0, CHUNK, unroll=4)
        def _edge(e):
            eidx = jnp.broadcast_to(e, (16,)).astype(jnp.int32)
            av = plsc.load_gather(alpha_v, [eidx])
            for k in range(4):
                prod[e, pl.ds(16 * k, 16)] = xlr[e, pl.ds(16 * k, 16)] * av

        pltpu.sync_copy(prod, osh.at[dst_v], add=True)
        return carry

    lax.fori_loop(0, CPT, chunk_body, 0)
    plsc.subcore_barrier()
    pltpu.sync_copy(osh.at[pl.ds(s * RPW, RPW)],
                    hpart_hbm.at[c, pl.ds(s * RPW, RPW)])


def _sc_scatter(xl, src, dst, ex, dpart):
    f = pl.kernel(
        _sc_scatter_body,
        out_type=[
            jax.ShapeDtypeStruct((NC, NPAD, C), jnp.float32),
        ],
        mesh=_MESH,
        compiler_params=pltpu.CompilerParams(
            needs_layout_passes=False, use_tc_tiling_on_sc=False),
        scratch_types=[
            pltpu.VMEM((CHUNK,), jnp.int32),
            pltpu.VMEM((CHUNK,), jnp.int32),
            pltpu.VMEM((CHUNK,), jnp.float32),
            pltpu.VMEM((CHUNK,), jnp.float32),
            pltpu.VMEM((CHUNK, C), jnp.float32),
            pltpu.VMEM((CHUNK, C), jnp.float32),
            pltpu.VMEM((NPAD,), jnp.float32),
            pltpu.VMEM((NPAD,), jnp.float32),
            pltpu.VMEM((C, C), jnp.float32),
            pltpu.VMEM_SHARED((NPAD, C), jnp.float32),
            pltpu.SemaphoreType.DMA,
        ],
    )
    return f(xl, src, dst, ex, dpart)[0]


def kernel(x, edge_index, Wl1, bl1, Wr1, br1, att1, bias1,
           Wl2, bl2, Wr2, br2, att2, bias2, Wd, bd):
    loop = jnp.arange(N, dtype=jnp.int32)
    src = jnp.concatenate([edge_index[0].astype(jnp.int32), loop])
    dst = jnp.concatenate([edge_index[1].astype(jnp.int32), loop])
    padi = jnp.arange(E_PAD - E_TOTAL, dtype=jnp.int32) % N
    src = jnp.concatenate([src, padi])
    dst = jnp.concatenate([dst, padi])

    att1f = att1.reshape(C)
    att2f = att2.reshape(C)
    r = lambda v: v.reshape(1, -1)

    xl1, xr1 = _dense_pair(x, Wl1, r(bl1), Wr1, r(br1))
    ex1, dp1 = _sc_logits(xl1, xr1, src, dst, att1f)
    hp1 = _sc_scatter(xl1, src, dst, ex1, dp1)

    xl2, xr2 = _relu_dense_pair(hp1[0, :N], hp1[1, :N], r(bias1),
                                Wl2, r(bl2), Wr2, r(br2))
    ex2, dp2 = _sc_logits(xl2, xr2, src, dst, att2f)
    hp2 = _sc_scatter(xl2, src, dst, ex2, dp2)

    h, out = _decode(hp2[0, :N], hp2[1, :N], r(bias2), Wd, r(bd))
    return (out, h)


# speedup vs baseline: 17.7025x; 1.4817x over previous
import jax
import jax.numpy as jnp
from jax import lax
from jax.experimental import pallas as pl
from jax.experimental.pallas import tpu as pltpu
from jax.experimental.pallas import tpu_sc as plsc

N = 10000
NPAD = 10240
C = 64
NC, NS, L = 2, 16, 16
NW = NC * NS
CHUNK = 128
E_TOTAL = 330000
CPT = 81
EPT = CPT * CHUNK
E_PAD = NW * EPT
RPW = NPAD // NS

_MESH = plsc.VectorSubcoreMesh(core_axis_name="c", subcore_axis_name="s")
_SC_PARAMS = pltpu.CompilerParams(
    needs_layout_passes=False, use_tc_tiling_on_sc=False)


def _proj_body(x_ref, wl_ref, bl_ref, wr_ref, br_ref, ol_ref, or_ref):
    xb = x_ref[...]
    ol_ref[...] = jnp.dot(xb, wl_ref[...],
                          preferred_element_type=jnp.float32) + bl_ref[...]
    or_ref[...] = jnp.dot(xb, wr_ref[...],
                          preferred_element_type=jnp.float32) + br_ref[...]


def _dense_pair(x, Wl, bl, Wr, br):
    n, f = x.shape
    c = Wl.shape[1]
    blk = 400
    return pl.pallas_call(
        _proj_body,
        grid=(n // blk,),
        in_specs=[
            pl.BlockSpec((blk, f), lambda i: (i, 0)),
            pl.BlockSpec((f, c), lambda i: (0, 0)),
            pl.BlockSpec((1, c), lambda i: (0, 0)),
            pl.BlockSpec((f, c), lambda i: (0, 0)),
            pl.BlockSpec((1, c), lambda i: (0, 0)),
        ],
        out_specs=[
            pl.BlockSpec((blk, c), lambda i: (i, 0)),
            pl.BlockSpec((blk, c), lambda i: (i, 0)),
        ],
        out_shape=[jax.ShapeDtypeStruct((n, c), jnp.float32)] * 2,
    )(x, Wl, bl, Wr, br)


def _norm_relu_proj_body(p0_ref, p1_ref, d0_ref, d1_ref, b_ref,
                         wl_ref, bl_ref, wr_ref, br_ref, ol_ref, or_ref):
    dn = d0_ref[...] + d1_ref[...] + 1e-16
    h = jnp.maximum((p0_ref[...] + p1_ref[...]) / dn + b_ref[...], 0.0)
    ol_ref[...] = jnp.dot(h, wl_ref[...],
                          preferred_element_type=jnp.float32) + bl_ref[...]
    or_ref[...] = jnp.dot(h, wr_ref[...],
                          preferred_element_type=jnp.float32) + br_ref[...]


def _norm_relu_dense_pair(p0, p1, d0, d1, b, Wl, bl, Wr, br):
    n, f = p0.shape
    c = Wl.shape[1]
    blk = 400
    return pl.pallas_call(
        _norm_relu_proj_body,
        grid=(n // blk,),
        in_specs=[
            pl.BlockSpec((blk, f), lambda i: (i, 0)),
            pl.BlockSpec((blk, f), lambda i: (i, 0)),
            pl.BlockSpec((blk, 1), lambda i: (i, 0)),
            pl.BlockSpec((blk, 1), lambda i: (i, 0)),
            pl.BlockSpec((1, f), lambda i: (0, 0)),
            pl.BlockSpec((f, c), lambda i: (0, 0)),
            pl.BlockSpec((1, c), lambda i: (0, 0)),
            pl.BlockSpec((f, c), lambda i: (0, 0)),
            pl.BlockSpec((1, c), lambda i: (0, 0)),
        ],
        out_specs=[
            pl.BlockSpec((blk, c), lambda i: (i, 0)),
            pl.BlockSpec((blk, c), lambda i: (i, 0)),
        ],
        out_shape=[jax.ShapeDtypeStruct((n, c), jnp.float32)] * 2,
    )(p0, p1, d0, d1, b, Wl, bl, Wr, br)


def _decode_body(p0_ref, p1_ref, d0_ref, d1_ref, b_ref, wd_ref, bd_ref,
                 h_ref, out_ref):
    dn = d0_ref[...] + d1_ref[...] + 1e-16
    h = (p0_ref[...] + p1_ref[...]) / dn + b_ref[...]
    h_ref[...] = h
    out_ref[...] = jnp.dot(h, wd_ref[...],
                           preferred_element_type=jnp.float32) + bd_ref[...]


def _decode(p0, p1, d0, d1, b, Wd, bd):
    n, f = p0.shape
    od = Wd.shape[1]
    blk = 400
    return pl.pallas_call(
        _decode_body,
        grid=(n // blk,),
        in_specs=[
            pl.BlockSpec((blk, f), lambda i: (i, 0)),
            pl.BlockSpec((blk, f), lambda i: (i, 0)),
            pl.BlockSpec((blk, 1), lambda i: (i, 0)),
            pl.BlockSpec((blk, 1), lambda i: (i, 0)),
            pl.BlockSpec((1, f), lambda i: (0, 0)),
            pl.BlockSpec((f, od), lambda i: (0, 0)),
            pl.BlockSpec((1, od), lambda i: (0, 0)),
        ],
        out_specs=[
            pl.BlockSpec((blk, f), lambda i: (i, 0)),
            pl.BlockSpec((blk, od), lambda i: (i, 0)),
        ],
        out_shape=[
            jax.ShapeDtypeStruct((n, f), jnp.float32),
            jax.ShapeDtypeStruct((n, od), jnp.float32),
        ],
    )(p0, p1, d0, d1, b, Wd, bd)


def _sc_layer_body(xl_hbm, xr_hbm, src_hbm, dst_hbm, att_hbm,
                   ppart_hbm, dpart_hbm,
                   src_v, dst_v, xlr, xrr, prod, att_v, lbuf, exbuf, zbuf,
                   osh, dsh, sem1, sem2):
    c = lax.axis_index("c")
    s = lax.axis_index("s")
    wid = c * NS + s

    for e in range(CHUNK):
        for k in range(C // L):
            prod[e, pl.ds(k * L, L)] = jnp.zeros((L,), jnp.float32)
    for i in range(RPW // CHUNK):
        pltpu.sync_copy(prod, osh.at[pl.ds(s * RPW + i * CHUNK, CHUNK)])
    for i in range(RPW // L):
        zbuf[pl.ds(i * L, L)] = jnp.zeros((L,), jnp.float32)
    pltpu.sync_copy(zbuf, dsh.at[pl.ds(s * RPW, RPW)])
    plsc.subcore_barrier()

    pltpu.sync_copy(att_hbm, att_v)
    att_regs = [att_v[pl.ds(16 * k, 16)] for k in range(4)]
    lane = lax.iota(jnp.int32, 16)
    tile_base = wid * EPT

    def chunk_body(ch, carry):
        base = tile_base + ch * CHUNK
        pltpu.sync_copy(src_hbm.at[pl.ds(base, CHUNK)], src_v)
        pltpu.sync_copy(dst_hbm.at[pl.ds(base, CHUNK)], dst_v)
        cp1 = pltpu.async_copy(xl_hbm.at[src_v], xlr, sem1)
        cp2 = pltpu.async_copy(xr_hbm.at[dst_v], xrr, sem2)
        cp1.wait()
        cp2.wait()

        @plsc.parallel_loop(0, CHUNK, unroll=4)
        def _edge(e):
            w = None
            for k in range(4):
                m = xlr[e, pl.ds(16 * k, 16)] + xrr[e, pl.ds(16 * k, 16)]
                t = jnp.maximum(m, 0.2 * m) * att_regs[k]
                w = t if w is None else w + t
            lbuf[e, :] = w

        for g in range(CHUNK // L):
            rows = L * g + lane
            acc = plsc.load_gather(lbuf, [rows, jnp.full((L,), 0, jnp.int32)])
            for l in range(1, L):
                acc = acc + plsc.load_gather(
                    lbuf, [rows, jnp.full((L,), l, jnp.int32)])
            gidx = base + L * g + lane
            ve = jnp.where(gidx < E_TOTAL,
                           jnp.exp(jnp.minimum(acc, 60.0)), 0.0)
            exbuf[pl.ds(L * g, L)] = ve

        @plsc.parallel_loop(0, CHUNK, unroll=4)
        def _edge2(e):
            eidx = jnp.broadcast_to(e, (16,)).astype(jnp.int32)
            exv = plsc.load_gather(exbuf, [eidx])
            for k in range(4):
                prod[e, pl.ds(16 * k, 16)] = xlr[e, pl.ds(16 * k, 16)] * exv

        pltpu.sync_copy(exbuf, dsh.at[dst_v], add=True)
        pltpu.sync_copy(prod, osh.at[dst_v], add=True)
        return carry

    lax.fori_loop(0, CPT, chunk_body, 0)
    plsc.subcore_barrier()

    pltpu.sync_copy(osh.at[pl.ds(s * RPW, RPW)],
                    ppart_hbm.at[c, pl.ds(s * RPW, RPW)])

    @pl.when(s == 0)
    def _():
        pltpu.sync_copy(dsh, dpart_hbm.at[c])


def _sc_layer(xl, xr, src, dst, att):
    f = pl.kernel(
        _sc_layer_body,
        out_type=[
            jax.ShapeDtypeStruct((NC, NPAD, C), jnp.float32),
            jax.ShapeDtypeStruct((NC, NPAD), jnp.float32),
        ],
        mesh=_MESH,
        compiler_params=_SC_PARAMS,
        scratch_types=[
            pltpu.VMEM((CHUNK,), jnp.int32),
            pltpu.VMEM((CHUNK,), jnp.int32),
            pltpu.VMEM((CHUNK, C), jnp.float32),
            pltpu.VMEM((CHUNK, C), jnp.float32),
            pltpu.VMEM((CHUNK, C), jnp.float32),
            pltpu.VMEM((C,), jnp.float32),
            pltpu.VMEM((CHUNK, L), jnp.float32),
            pltpu.VMEM((CHUNK,), jnp.float32),
            pltpu.VMEM((RPW,), jnp.float32),
            pltpu.VMEM_SHARED((NPAD, C), jnp.float32),
            pltpu.VMEM_SHARED((NPAD,), jnp.float32),
            pltpu.SemaphoreType.DMA,
            pltpu.SemaphoreType.DMA,
        ],
    )
    return f(xl, xr, src, dst, att)


def kernel(x, edge_index, Wl1, bl1, Wr1, br1, att1, bias1,
           Wl2, bl2, Wr2, br2, att2, bias2, Wd, bd):
    loop = jnp.arange(N, dtype=jnp.int32)
    src = jnp.concatenate([edge_index[0].astype(jnp.int32), loop])
    dst = jnp.concatenate([edge_index[1].astype(jnp.int32), loop])
    padi = jnp.arange(E_PAD - E_TOTAL, dtype=jnp.int32) % N
    src = jnp.concatenate([src, padi])
    dst = jnp.concatenate([dst, padi])

    att1f = att1.reshape(C)
    att2f = att2.reshape(C)
    r = lambda v: v.reshape(1, -1)

    xl1, xr1 = _dense_pair(x, Wl1, r(bl1), Wr1, r(br1))
    pp1, dp1 = _sc_layer(xl1, xr1, src, dst, att1f)

    xl2, xr2 = _norm_relu_dense_pair(
        pp1[0, :N], pp1[1, :N], dp1[0, :N, None], dp1[1, :N, None],
        r(bias1), Wl2, r(bl2), Wr2, r(br2))
    pp2, dp2 = _sc_layer(xl2, xr2, src, dst, att2f)

    h, out = _decode(pp2[0, :N], pp2[1, :N], dp2[0, :N, None],
                     dp2[1, :N, None], r(bias2), Wd, r(bd))
    return (out, h)


# speedup vs baseline: 28.2418x; 1.5954x over previous
import jax
import jax.numpy as jnp
from jax import lax
from jax.experimental import pallas as pl
from jax.experimental.pallas import tpu as pltpu
from jax.experimental.pallas import tpu_sc as plsc

N = 10000
NPAD = 10240
C = 64
NC, NS, L = 2, 16, 16
NW = NC * NS
CHUNK = 128
E_TOTAL = 330000
CPT = 82
EPT = CPT * CHUNK
E_PAD = NW * EPT
RPW = NPAD // NS

_MESH = plsc.VectorSubcoreMesh(core_axis_name="c", subcore_axis_name="s")
_SC_PARAMS = pltpu.CompilerParams(
    needs_layout_passes=False, use_tc_tiling_on_sc=False)


def _proj_body(x_ref, wl_ref, bl_ref, wr_ref, br_ref, ol_ref, or_ref):
    xb = x_ref[...]
    ol_ref[...] = jnp.dot(xb, wl_ref[...],
                          preferred_element_type=jnp.float32) + bl_ref[...]
    or_ref[...] = jnp.dot(xb, wr_ref[...],
                          preferred_element_type=jnp.float32) + br_ref[...]


def _dense_pair(x, Wl, bl, Wr, br):
    n, f = x.shape
    c = Wl.shape[1]
    blk = 400
    return pl.pallas_call(
        _proj_body,
        grid=(n // blk,),
        in_specs=[
            pl.BlockSpec((blk, f), lambda i: (i, 0)),
            pl.BlockSpec((f, c), lambda i: (0, 0)),
            pl.BlockSpec((1, c), lambda i: (0, 0)),
            pl.BlockSpec((f, c), lambda i: (0, 0)),
            pl.BlockSpec((1, c), lambda i: (0, 0)),
        ],
        out_specs=[
            pl.BlockSpec((blk, c), lambda i: (i, 0)),
            pl.BlockSpec((blk, c), lambda i: (i, 0)),
        ],
        out_shape=[jax.ShapeDtypeStruct((n, c), jnp.float32)] * 2,
    )(x, Wl, bl, Wr, br)


def _norm_relu_proj_body(p0_ref, p1_ref, d0_ref, d1_ref, b_ref,
                         wl_ref, bl_ref, wr_ref, br_ref, ol_ref, or_ref):
    dn = d0_ref[...] + d1_ref[...] + 1e-16
    h = jnp.maximum((p0_ref[...] + p1_ref[...]) / dn + b_ref[...], 0.0)
    ol_ref[...] = jnp.dot(h, wl_ref[...],
                          preferred_element_type=jnp.float32) + bl_ref[...]
    or_ref[...] = jnp.dot(h, wr_ref[...],
                          preferred_element_type=jnp.float32) + br_ref[...]


def _norm_relu_dense_pair(p0, p1, d0, d1, b, Wl, bl, Wr, br):
    n, f = p0.shape
    c = Wl.shape[1]
    blk = 400
    return pl.pallas_call(
        _norm_relu_proj_body,
        grid=(n // blk,),
        in_specs=[
            pl.BlockSpec((blk, f), lambda i: (i, 0)),
            pl.BlockSpec((blk, f), lambda i: (i, 0)),
            pl.BlockSpec((blk, 1), lambda i: (i, 0)),
            pl.BlockSpec((blk, 1), lambda i: (i, 0)),
            pl.BlockSpec((1, f), lambda i: (0, 0)),
            pl.BlockSpec((f, c), lambda i: (0, 0)),
            pl.BlockSpec((1, c), lambda i: (0, 0)),
            pl.BlockSpec((f, c), lambda i: (0, 0)),
            pl.BlockSpec((1, c), lambda i: (0, 0)),
        ],
        out_specs=[
            pl.BlockSpec((blk, c), lambda i: (i, 0)),
            pl.BlockSpec((blk, c), lambda i: (i, 0)),
        ],
        out_shape=[jax.ShapeDtypeStruct((n, c), jnp.float32)] * 2,
    )(p0, p1, d0, d1, b, Wl, bl, Wr, br)


def _decode_body(p0_ref, p1_ref, d0_ref, d1_ref, b_ref, wd_ref, bd_ref,
                 h_ref, out_ref):
    dn = d0_ref[...] + d1_ref[...] + 1e-16
    h = (p0_ref[...] + p1_ref[...]) / dn + b_ref[...]
    h_ref[...] = h
    out_ref[...] = jnp.dot(h, wd_ref[...],
                           preferred_element_type=jnp.float32) + bd_ref[...]


def _decode(p0, p1, d0, d1, b, Wd, bd):
    n, f = p0.shape
    od = Wd.shape[1]
    blk = 400
    return pl.pallas_call(
        _decode_body,
        grid=(n // blk,),
        in_specs=[
            pl.BlockSpec((blk, f), lambda i: (i, 0)),
            pl.BlockSpec((blk, f), lambda i: (i, 0)),
            pl.BlockSpec((blk, 1), lambda i: (i, 0)),
            pl.BlockSpec((blk, 1), lambda i: (i, 0)),
            pl.BlockSpec((1, f), lambda i: (0, 0)),
            pl.BlockSpec((f, od), lambda i: (0, 0)),
            pl.BlockSpec((1, od), lambda i: (0, 0)),
        ],
        out_specs=[
            pl.BlockSpec((blk, f), lambda i: (i, 0)),
            pl.BlockSpec((blk, od), lambda i: (i, 0)),
        ],
        out_shape=[
            jax.ShapeDtypeStruct((n, f), jnp.float32),
            jax.ShapeDtypeStruct((n, od), jnp.float32),
        ],
    )(p0, p1, d0, d1, b, Wd, bd)


def _sc_layer_body(xl_hbm, xr_hbm, src_hbm, dst_hbm, att_hbm,
                   ppart_hbm, dpart_hbm,
                   src_all, dst_all, xlr0, xrr0, xlr1, xrr1, prod, att_v,
                   lbuf, exbuf, zbuf,
                   osh, dsh, sl0, sr0, sl1, sr1):
    c = lax.axis_index("c")
    s = lax.axis_index("s")
    wid = c * NS + s

    for e in range(CHUNK):
        for k in range(C // L):
            prod[e, pl.ds(k * L, L)] = jnp.zeros((L,), jnp.float32)
    for i in range(RPW // CHUNK):
        pltpu.sync_copy(prod, osh.at[pl.ds(s * RPW + i * CHUNK, CHUNK)])
    for i in range(RPW // L):
        zbuf[pl.ds(i * L, L)] = jnp.zeros((L,), jnp.float32)
    pltpu.sync_copy(zbuf, dsh.at[pl.ds(s * RPW, RPW)])
    plsc.subcore_barrier()

    pltpu.sync_copy(att_hbm, att_v)
    pltpu.sync_copy(src_hbm.at[pl.ds(wid * CPT, CPT)], src_all)
    pltpu.sync_copy(dst_hbm.at[pl.ds(wid * CPT, CPT)], dst_all)
    att_regs = [att_v[pl.ds(16 * k, 16)] for k in range(4)]
    lane = lax.iota(jnp.int32, 16)
    tile_base = wid * EPT

    bufs = [(xlr0, xrr0, sl0, sr0), (xlr1, xrr1, sl1, sr1)]

    def start_rows(cb, b):
        xlr, xrr, sl, sr = bufs[b]
        pltpu.async_copy(xl_hbm.at[src_all.at[cb]], xlr, sl)
        pltpu.async_copy(xr_hbm.at[dst_all.at[cb]], xrr, sr)

    def wait_rows(cb, b):
        xlr, xrr, sl, sr = bufs[b]
        pltpu.make_async_copy(xl_hbm.at[src_all.at[cb]], xlr, sl).wait()
        pltpu.make_async_copy(xr_hbm.at[dst_all.at[cb]], xrr, sr).wait()

    def compute(cb, b):
        xlr, xrr, _, _ = bufs[b]
        base = tile_base + cb * CHUNK

        @plsc.parallel_loop(0, CHUNK, unroll=4)
        def _edge(e):
            w = None
            for k in range(4):
                m = xlr[e, pl.ds(16 * k, 16)] + xrr[e, pl.ds(16 * k, 16)]
                t = jnp.maximum(m, 0.2 * m) * att_regs[k]
                w = t if w is None else w + t
            lbuf[e, :] = w

        for g in range(CHUNK // L):
            rows = L * g + lane
            acc = plsc.load_gather(lbuf, [rows, jnp.full((L,), 0, jnp.int32)])
            for l in range(1, L):
                acc = acc + plsc.load_gather(
                    lbuf, [rows, jnp.full((L,), l, jnp.int32)])
            gidx = base + L * g + lane
            ve = jnp.where(gidx < E_TOTAL,
                           jnp.exp(jnp.minimum(acc, 60.0)), 0.0)
            exbuf[pl.ds(L * g, L)] = ve

        @plsc.parallel_loop(0, CHUNK, unroll=4)
        def _edge2(e):
            eidx = jnp.broadcast_to(e, (16,)).astype(jnp.int32)
            exv = plsc.load_gather(exbuf, [eidx])
            for k in range(4):
                prod[e, pl.ds(16 * k, 16)] = xlr[e, pl.ds(16 * k, 16)] * exv

        pltpu.sync_copy(exbuf, dsh.at[dst_all.at[cb]], add=True)
        pltpu.sync_copy(prod, osh.at[dst_all.at[cb]], add=True)

    start_rows(0, 0)

    @pl.loop(0, CPT, step=2)
    def _pair(ch):
        start_rows(ch + 1, 1)
        wait_rows(ch, 0)
        compute(ch, 0)

        @pl.when(ch + 2 < CPT)
        def _():
            start_rows(ch + 2, 0)

        wait_rows(ch + 1, 1)
        compute(ch + 1, 1)

    plsc.subcore_barrier()

    pltpu.sync_copy(osh.at[pl.ds(s * RPW, RPW)],
                    ppart_hbm.at[c, pl.ds(s * RPW, RPW)])

    @pl.when(s == 0)
    def _():
        pltpu.sync_copy(dsh, dpart_hbm.at[c])


def _sc_layer(xl, xr, src, dst, att):
    f = pl.kernel(
        _sc_layer_body,
        out_type=[
            jax.ShapeDtypeStruct((NC, NPAD, C), jnp.float32),
            jax.ShapeDtypeStruct((NC, NPAD), jnp.float32),
        ],
        mesh=_MESH,
        compiler_params=_SC_PARAMS,
        scratch_types=[
            pltpu.VMEM((CPT, CHUNK), jnp.int32),
            pltpu.VMEM((CPT, CHUNK), jnp.int32),
            pltpu.VMEM((CHUNK, C), jnp.float32),
            pltpu.VMEM((CHUNK, C), jnp.float32),
            pltpu.VMEM((CHUNK, C), jnp.float32),
            pltpu.VMEM((CHUNK, C), jnp.float32),
            pltpu.VMEM((CHUNK, C), jnp.float32),
            pltpu.VMEM((C,), jnp.float32),
            pltpu.VMEM((CHUNK, L), jnp.float32),
            pltpu.VMEM((CHUNK,), jnp.float32),
            pltpu.VMEM((RPW,), jnp.float32),
            pltpu.VMEM_SHARED((NPAD, C), jnp.float32),
            pltpu.VMEM_SHARED((NPAD,), jnp.float32),
            pltpu.SemaphoreType.DMA,
            pltpu.SemaphoreType.DMA,
            pltpu.SemaphoreType.DMA,
            pltpu.SemaphoreType.DMA,
        ],
    )
    return f(xl, xr, src, dst, att)


def kernel(x, edge_index, Wl1, bl1, Wr1, br1, att1, bias1,
           Wl2, bl2, Wr2, br2, att2, bias2, Wd, bd):
    loop = jnp.arange(N, dtype=jnp.int32)
    src = jnp.concatenate([edge_index[0].astype(jnp.int32), loop])
    dst = jnp.concatenate([edge_index[1].astype(jnp.int32), loop])
    padi = jnp.arange(E_PAD - E_TOTAL, dtype=jnp.int32) % N
    src = jnp.concatenate([src, padi]).reshape(NW * CPT, CHUNK)
    dst = jnp.concatenate([dst, padi]).reshape(NW * CPT, CHUNK)

    att1f = att1.reshape(C)
    att2f = att2.reshape(C)
    r = lambda v: v.reshape(1, -1)

    xl1, xr1 = _dense_pair(x, Wl1, r(bl1), Wr1, r(br1))
    pp1, dp1 = _sc_layer(xl1, xr1, src, dst, att1f)

    xl2, xr2 = _norm_relu_dense_pair(
        pp1[0, :N], pp1[1, :N], dp1[0, :N, None], dp1[1, :N, None],
        r(bias1), Wl2, r(bl2), Wr2, r(br2))
    pp2, dp2 = _sc_layer(xl2, xr2, src, dst, att2f)

    h, out = _decode(pp2[0, :N], pp2[1, :N], dp2[0, :N, None],
                     dp2[1, :N, None], r(bias2), Wd, r(bd))
    return (out, h)
